# Initial kernel scaffold; baseline (speedup 1.0000x reference)
import jax
import jax.numpy as jnp
from jax.experimental import pallas as pl


def kernel(triplet_idx, entity_embedding, relation_embedding, w_vector):
    raise NotImplementedError("write your pallas kernel here")


# speedup vs baseline: 1.0108x; 1.0108x over previous
import functools

import jax
import jax.numpy as jnp
from jax import lax
from jax.experimental import pallas as pl
from jax.experimental.pallas import tpu as pltpu
from jax.experimental.pallas import tpu_sc as plsc

BATCH = 16384
DIM = 128
CHUNK = 128
LANES = 16

_INFO = plsc.get_sparse_core_info()
_NC = _INFO.num_cores
_NS = _INFO.num_subcores
_NW = _NC * _NS
_BPW = BATCH // _NW
_NCHUNK = _BPW // CHUNK


def _rsqrt_nr(x):
    i = plsc.bitcast(x, jnp.int32)
    i = 0x5F3759DF - lax.shift_right_logical(i, 1)
    y = plsc.bitcast(i, jnp.float32)
    for _ in range(3):
        y = y * (1.5 - 0.5 * x * y * y)
    return y


def _sc_body(ent_hbm, rel_hbm, w_hbm, ih_hbm, ir_hbm, it_hbm, out_hbm,
             ih_v, ir_v, it_v, head_v, tail_v, rel_v, w_v, score_v, sem):
    wid = lax.axis_index("s") * _NC + lax.axis_index("c")
    base = wid * _BPW
    tiny = 1e-24

    for c in range(_NCHUNK):
        cbase = base + c * CHUNK
        pltpu.sync_copy(ih_hbm.at[pl.ds(cbase, CHUNK)], ih_v)
        pltpu.sync_copy(ir_hbm.at[pl.ds(cbase, CHUNK)], ir_v)
        pltpu.sync_copy(it_hbm.at[pl.ds(cbase, CHUNK)], it_v)
        cp_h = pltpu.async_copy(ent_hbm.at[ih_v], head_v, sem)
        cp_t = pltpu.async_copy(ent_hbm.at[it_v], tail_v, sem)
        cp_r = pltpu.async_copy(rel_hbm.at[ir_v], rel_v, sem)
        cp_w = pltpu.async_copy(w_hbm.at[ir_v], w_v, sem)
        cp_h.wait()
        cp_t.wait()
        cp_r.wait()
        cp_w.wait()

        for g in range(CHUNK // LANES):
            rows = lax.iota(jnp.int32, LANES) + (g * LANES)

            def jbody(j, acc):
                aa, aw, ar, dw, ww, rr, rw = acc
                cols = jnp.full((LANES,), j, dtype=jnp.int32)
                h = plsc.load_gather(head_v, [rows, cols])
                t = plsc.load_gather(tail_v, [rows, cols])
                r = plsc.load_gather(rel_v, [rows, cols])
                w = plsc.load_gather(w_v, [rows, cols])
                d = h - t
                a = d + 1e-6
                return (aa + a * a, aw + a * w, ar + a * r, dw + d * w,
                        ww + w * w, rr + r * r, rw + r * w)

            zeros = jnp.zeros((LANES,), jnp.float32)
            aa, aw, ar, dw, ww, rr, rw = lax.fori_loop(
                0, DIM, jbody, (zeros,) * 7)

            g_ = dw / jnp.maximum(ww, tiny)
            b_ = _rsqrt_nr(jnp.maximum(rr, tiny))
            val = (aa + b_ * b_ * rr + 2.0 * b_ * ar - 2.0 * g_ * aw
                   - 2.0 * b_ * g_ * rw + g_ * g_ * ww)
            val = jnp.maximum(val, 0.0)
            score = val * _rsqrt_nr(jnp.maximum(val, tiny))
            score_v[pl.ds(g * LANES, LANES)] = score

        pltpu.sync_copy(score_v, out_hbm.at[pl.ds(cbase, CHUNK)])


@jax.jit
def _transh_sc(ent, rel, w, ih, ir, it):
    mesh = plsc.VectorSubcoreMesh(core_axis_name="c", subcore_axis_name="s")
    f = functools.partial(
        pl.kernel,
        out_type=jax.ShapeDtypeStruct((BATCH,), jnp.float32),
        mesh=mesh,
        compiler_params=pltpu.CompilerParams(needs_layout_passes=False),
        scratch_types=[
            pltpu.VMEM((CHUNK,), jnp.int32),
            pltpu.VMEM((CHUNK,), jnp.int32),
            pltpu.VMEM((CHUNK,), jnp.int32),
            pltpu.VMEM((CHUNK, DIM), jnp.float32),
            pltpu.VMEM((CHUNK, DIM), jnp.float32),
            pltpu.VMEM((CHUNK, DIM), jnp.float32),
            pltpu.VMEM((CHUNK, DIM), jnp.float32),
            pltpu.VMEM((CHUNK,), jnp.float32),
            pltpu.SemaphoreType.DMA,
        ],
    )(_sc_body)
    return f(ent, rel, w, ih, ir, it)


def kernel(triplet_idx, entity_embedding, relation_embedding, w_vector):
    idx = triplet_idx.astype(jnp.int32)
    ih = jnp.asarray(idx[:, 0])
    ir = jnp.asarray(idx[:, 1])
    it = jnp.asarray(idx[:, 2])
    return _transh_sc(entity_embedding, relation_embedding, w_vector,
                      ih, ir, it)


# speedup vs baseline: 2.8362x; 2.8060x over previous
import functools

import jax
import jax.numpy as jnp
from jax import lax
from jax.experimental import pallas as pl
from jax.experimental.pallas import tpu as pltpu
from jax.experimental.pallas import tpu_sc as plsc

BATCH = 16384
DIM = 128
CHUNK = 128
LANES = 16

_INFO = plsc.get_sparse_core_info()
_NC = _INFO.num_cores
_NS = _INFO.num_subcores
_NW = _NC * _NS
_BPW = BATCH // _NW
_NCHUNK = _BPW // CHUNK


def _rsqrt_nr(x):
    i = plsc.bitcast(x, jnp.int32)
    i = 0x5F3759DF - lax.shift_right_logical(i, 1)
    y = plsc.bitcast(i, jnp.float32)
    for _ in range(3):
        y = y * (1.5 - 0.5 * x * y * y)
    return y


def _sc_body(ent_hbm, rel_hbm, w_hbm, ih_hbm, ir_hbm, it_hbm, out_hbm,
             ih_v, ir_v, it_v, head_v, tail_v, rel_v, w_v, score_v, sem):
    wid = lax.axis_index("s") * _NC + lax.axis_index("c")
    base = wid * _BPW
    tiny = 1e-24

    for c in range(_NCHUNK):
        cbase = base + c * CHUNK
        pltpu.sync_copy(ih_hbm.at[pl.ds(cbase, CHUNK)], ih_v)
        pltpu.sync_copy(ir_hbm.at[pl.ds(cbase, CHUNK)], ir_v)
        pltpu.sync_copy(it_hbm.at[pl.ds(cbase, CHUNK)], it_v)
        cp_h = pltpu.async_copy(ent_hbm.at[ih_v], head_v, sem)
        cp_t = pltpu.async_copy(ent_hbm.at[it_v], tail_v, sem)
        cp_r = pltpu.async_copy(rel_hbm.at[ir_v], rel_v, sem)
        cp_w = pltpu.async_copy(w_hbm.at[ir_v], w_v, sem)
        cp_h.wait()
        cp_t.wait()
        cp_r.wait()
        cp_w.wait()

        for g in range(CHUNK // LANES):
            lane = lax.iota(jnp.int32, LANES)
            rows = lane + (g * LANES)

            def jbody(j, acc):
                aa, aw, ar, dw, ww, rr, rw = acc
                cols = jnp.bitwise_and(lane + j, DIM - 1)
                h = plsc.load_gather(head_v, [rows, cols])
                t = plsc.load_gather(tail_v, [rows, cols])
                r = plsc.load_gather(rel_v, [rows, cols])
                w = plsc.load_gather(w_v, [rows, cols])
                d = h - t
                a = d + 1e-6
                return (aa + a * a, aw + a * w, ar + a * r, dw + d * w,
                        ww + w * w, rr + r * r, rw + r * w)

            zeros = jnp.zeros((LANES,), jnp.float32)
            aa, aw, ar, dw, ww, rr, rw = lax.fori_loop(
                0, DIM, jbody, (zeros,) * 7)

            g_ = dw / jnp.maximum(ww, tiny)
            b_ = _rsqrt_nr(jnp.maximum(rr, tiny))
            val = (aa + b_ * b_ * rr + 2.0 * b_ * ar - 2.0 * g_ * aw
                   - 2.0 * b_ * g_ * rw + g_ * g_ * ww)
            val = jnp.maximum(val, 0.0)
            score = val * _rsqrt_nr(jnp.maximum(val, tiny))
            score_v[pl.ds(g * LANES, LANES)] = score

        pltpu.sync_copy(score_v, out_hbm.at[pl.ds(cbase, CHUNK)])


@jax.jit
def _transh_sc(ent, rel, w, ih, ir, it):
    mesh = plsc.VectorSubcoreMesh(core_axis_name="c", subcore_axis_name="s")
    f = functools.partial(
        pl.kernel,
        out_type=jax.ShapeDtypeStruct((BATCH,), jnp.float32),
        mesh=mesh,
        compiler_params=pltpu.CompilerParams(needs_layout_passes=False),
        scratch_types=[
            pltpu.VMEM((CHUNK,), jnp.int32),
            pltpu.VMEM((CHUNK,), jnp.int32),
            pltpu.VMEM((CHUNK,), jnp.int32),
            pltpu.VMEM((CHUNK, DIM), jnp.float32),
            pltpu.VMEM((CHUNK, DIM), jnp.float32),
            pltpu.VMEM((CHUNK, DIM), jnp.float32),
            pltpu.VMEM((CHUNK, DIM), jnp.float32),
            pltpu.VMEM((CHUNK,), jnp.float32),
            pltpu.SemaphoreType.DMA,
        ],
    )(_sc_body)
    return f(ent, rel, w, ih, ir, it)


def kernel(triplet_idx, entity_embedding, relation_embedding, w_vector):
    idx = triplet_idx.astype(jnp.int32)
    ih = jnp.asarray(idx[:, 0])
    ir = jnp.asarray(idx[:, 1])
    it = jnp.asarray(idx[:, 2])
    return _transh_sc(entity_embedding, relation_embedding, w_vector,
                      ih, ir, it)


# speedup vs baseline: 3.7973x; 1.3389x over previous
import functools

import jax
import jax.numpy as jnp
from jax import lax
from jax.experimental import pallas as pl
from jax.experimental.pallas import tpu as pltpu
from jax.experimental.pallas import tpu_sc as plsc

BATCH = 16384
DIM = 128
CHUNK = 64
LANES = 16

_INFO = plsc.get_sparse_core_info()
_NC = _INFO.num_cores
_NS = _INFO.num_subcores
_NW = _NC * _NS
_BPW = BATCH // _NW
_NCHUNK = _BPW // CHUNK


def _rsqrt_nr(x):
    i = plsc.bitcast(x, jnp.int32)
    i = 0x5F3759DF - lax.shift_right_logical(i, 1)
    y = plsc.bitcast(i, jnp.float32)
    for _ in range(3):
        y = y * (1.5 - 0.5 * x * y * y)
    return y


def _sc_body(ent_hbm, rel_hbm, w_hbm, ih_hbm, ir_hbm, it_hbm, out_hbm,
             ih_v, ir_v, it_v, bufs, score_v, sems):
    wid = lax.axis_index("s") * _NC + lax.axis_index("c")
    base = wid * _BPW
    tiny = 1e-24

    pltpu.sync_copy(ih_hbm.at[pl.ds(base, _BPW)], ih_v)
    pltpu.sync_copy(ir_hbm.at[pl.ds(base, _BPW)], ir_v)
    pltpu.sync_copy(it_hbm.at[pl.ds(base, _BPW)], it_v)

    def fire(c, s):
        head_v, tail_v, rel_v, w_v = bufs[s]
        ih = ih_v.at[pl.ds(c * CHUNK, CHUNK)]
        ir = ir_v.at[pl.ds(c * CHUNK, CHUNK)]
        it = it_v.at[pl.ds(c * CHUNK, CHUNK)]
        return (pltpu.async_copy(ent_hbm.at[ih], head_v, sems[s]),
                pltpu.async_copy(ent_hbm.at[it], tail_v, sems[s]),
                pltpu.async_copy(rel_hbm.at[ir], rel_v, sems[s]),
                pltpu.async_copy(w_hbm.at[ir], w_v, sems[s]))

    pending = fire(0, 0)
    for c in range(_NCHUNK):
        s = c % 2
        nxt = fire(c + 1, 1 - s) if c + 1 < _NCHUNK else None
        for cp in pending:
            cp.wait()
        pending = nxt
        head_v, tail_v, rel_v, w_v = bufs[s]

        for g in range(CHUNK // LANES):
            lane = lax.iota(jnp.int32, LANES)
            rows = lane + (g * LANES)

            def jbody(j, acc):
                aa, aw, ar, dw, ww, rr, rw = acc
                cols = jnp.bitwise_and(lane + j, DIM - 1)
                h = plsc.load_gather(head_v, [rows, cols])
                t = plsc.load_gather(tail_v, [rows, cols])
                r = plsc.load_gather(rel_v, [rows, cols])
                w = plsc.load_gather(w_v, [rows, cols])
                d = h - t
                a = d + 1e-6
                return (aa + a * a, aw + a * w, ar + a * r, dw + d * w,
                        ww + w * w, rr + r * r, rw + r * w)

            zeros = jnp.zeros((LANES,), jnp.float32)
            aa, aw, ar, dw, ww, rr, rw = lax.fori_loop(
                0, DIM, jbody, (zeros,) * 7)

            g_ = dw / jnp.maximum(ww, tiny)
            b_ = _rsqrt_nr(jnp.maximum(rr, tiny))
            val = (aa + b_ * b_ * rr + 2.0 * b_ * ar - 2.0 * g_ * aw
                   - 2.0 * b_ * g_ * rw + g_ * g_ * ww)
            val = jnp.maximum(val, 0.0)
            score = val * _rsqrt_nr(jnp.maximum(val, tiny))
            score_v[pl.ds(c * CHUNK + g * LANES, LANES)] = score

    pltpu.sync_copy(score_v, out_hbm.at[pl.ds(base, _BPW)])


def _body_wrap(ent_hbm, rel_hbm, w_hbm, ih_hbm, ir_hbm, it_hbm, out_hbm,
               ih_v, ir_v, it_v,
               h0, t0, r0, w0, h1, t1, r1, w1, score_v, sem0, sem1):
    _sc_body(ent_hbm, rel_hbm, w_hbm, ih_hbm, ir_hbm, it_hbm, out_hbm,
             ih_v, ir_v, it_v,
             ((h0, t0, r0, w0), (h1, t1, r1, w1)), score_v, (sem0, sem1))


@jax.jit
def _transh_sc(ent, rel, w, ih, ir, it):
    mesh = plsc.VectorSubcoreMesh(core_axis_name="c", subcore_axis_name="s")
    buf = pltpu.VMEM((CHUNK, DIM), jnp.float32)
    f = functools.partial(
        pl.kernel,
        out_type=jax.ShapeDtypeStruct((BATCH,), jnp.float32),
        mesh=mesh,
        compiler_params=pltpu.CompilerParams(needs_layout_passes=False),
        scratch_types=[
            pltpu.VMEM((_BPW,), jnp.int32),
            pltpu.VMEM((_BPW,), jnp.int32),
            pltpu.VMEM((_BPW,), jnp.int32),
            buf, buf, buf, buf, buf, buf, buf, buf,
            pltpu.VMEM((_BPW,), jnp.float32),
            pltpu.SemaphoreType.DMA,
            pltpu.SemaphoreType.DMA,
        ],
    )(_body_wrap)
    return f(ent, rel, w, ih, ir, it)


def kernel(triplet_idx, entity_embedding, relation_embedding, w_vector):
    idx = triplet_idx.astype(jnp.int32)
    ih = jnp.asarray(idx[:, 0])
    ir = jnp.asarray(idx[:, 1])
    it = jnp.asarray(idx[:, 2])
    return _transh_sc(entity_embedding, relation_embedding, w_vector,
                      ih, ir, it)
